# baseline (speedup 1.0000x reference)
import functools

import jax
import jax.numpy as jnp
from jax import lax
from jax.experimental import pallas as pl
from jax.experimental.pallas import tpu as pltpu
from jax.experimental.pallas import tpu_sc as plsc

N = 10000
E = 320000
D = 128
HH = 128
C = 10
G = 64

NC = 2
NS = 16
NW = NC * NS
L = 16

EPT = E // NW
B_E = 80
NCHUNK = EPT // B_E
NPAD = 10240
RPT = 640
RPT_LAST = N - (NS - 1) * RPT

NBUF = 2

BLK = 1000
NB = N // BLK

_P = jax.lax.Precision.HIGHEST


def _sc_degree(dst_flat):
    mesh = plsc.VectorSubcoreMesh(core_axis_name="c", subcore_axis_name="s")

    @functools.partial(
        pl.kernel,
        out_type=jax.ShapeDtypeStruct((NW, NPAD), jnp.float32),
        mesh=mesh,
        scratch_types=[
            pltpu.VMEM((EPT,), jnp.int32),
            pltpu.VMEM((NPAD,), jnp.float32),
        ],
        compiler_params=pltpu.CompilerParams(needs_layout_passes=False),
    )
    def k(dst_hbm, out_hbm, dst_v, deg_v):
        cid = lax.axis_index("c")
        sid = lax.axis_index("s")
        w = cid * NS + sid
        pltpu.sync_copy(dst_hbm.at[pl.ds(w * EPT, EPT)], dst_v)

        zeros16 = jnp.zeros((L,), jnp.float32)

        def zbody(i, carry):
            deg_v[pl.ds(i * L, L)] = zeros16
            return carry

        lax.fori_loop(0, NPAD // L, zbody, 0)

        ones16 = jnp.ones((L,), jnp.float32)

        def sbody(i, carry):
            idx = dst_v[pl.ds(i * L, L)]
            plsc.addupdate_scatter(deg_v, [idx], ones16)
            return carry

        lax.fori_loop(0, EPT // L, sbody, 0)
        pltpu.sync_copy(deg_v, out_hbm.at[w])

    return k(dst_flat)


def _sc_scatter(y, src_flat, dst3):
    mesh = plsc.VectorSubcoreMesh(core_axis_name="c", subcore_axis_name="s")

    @functools.partial(
        pl.kernel,
        out_type=jax.ShapeDtypeStruct((NC * N, D), jnp.float32),
        mesh=mesh,
        scratch_types=[
            pltpu.VMEM((EPT,), jnp.int32),
            pltpu.VMEM((NCHUNK, B_E), jnp.int32),
            [pltpu.VMEM((B_E, D), jnp.float32) for _ in range(NBUF)],
            pltpu.VMEM_SHARED((N, D), jnp.float32),
            [pltpu.SemaphoreType.DMA for _ in range(NBUF)],
        ],
    )
    def k(y_hbm, src_hbm, dst_hbm, out_hbm, src_v, dst_v, rows_v,
          agg_sh, sems):
        cid = lax.axis_index("c")
        sid = lax.axis_index("s")
        w = cid * NS + sid

        pltpu.sync_copy(src_hbm.at[pl.ds(w * EPT, EPT)], src_v)
        pltpu.sync_copy(dst_hbm.at[w], dst_v)

        zeros16 = jnp.zeros((L,), jnp.float32)

        def zb(i, carry):
            for j in range(D // L):
                rows_v[0][i, pl.ds(j * L, L)] = zeros16
            return carry

        lax.fori_loop(0, B_E, zb, 0)

        def zs(i, carry):
            pltpu.sync_copy(rows_v[0],
                            agg_sh.at[pl.ds(sid * RPT + i * B_E, B_E)])
            return carry

        @pl.when(sid < NS - 1)
        def _():
            lax.fori_loop(0, RPT // B_E, zs, 0)

        @pl.when(sid == NS - 1)
        def _():
            lax.fori_loop(0, RPT_LAST // B_E, zs, 0)

        def gather(c, b):
            pltpu.async_copy(y_hbm.at[src_v.at[pl.ds(c * B_E, B_E)]],
                             rows_v[b], sems[b])

        for b in range(NBUF):
            gather(b, b)
        plsc.subcore_barrier()

        def wait_scatter(c, b):
            pltpu.make_async_copy(y_hbm.at[src_v.at[pl.ds(c * B_E, B_E)]],
                                  rows_v[b], sems[b]).wait()
            pltpu.sync_copy(rows_v[b], agg_sh.at[dst_v.at[c]], add=True)

        def body(i, carry):
            for b in range(NBUF):
                c = i * NBUF + b
                wait_scatter(c, b)
                gather(c + NBUF, b)
            return carry

        steady = (NCHUNK - NBUF) // NBUF
        lax.fori_loop(0, steady, body, 0)
        for b in range(NBUF):
            wait_scatter(steady * NBUF + b, b)
        for c in range((steady + 1) * NBUF, NCHUNK):
            b = c % NBUF
            gather(c, b)
            wait_scatter(c, b)
        plsc.subcore_barrier()

        @pl.when(sid < NS - 1)
        def _():
            pltpu.sync_copy(agg_sh.at[pl.ds(sid * RPT, RPT)],
                            out_hbm.at[pl.ds(cid * N + sid * RPT, RPT)])

        @pl.when(sid == NS - 1)
        def _():
            pltpu.sync_copy(
                agg_sh.at[pl.ds(sid * RPT, RPT_LAST)],
                out_hbm.at[pl.ds(cid * N + sid * RPT, RPT_LAST)])

    return k(y, src_flat, dst3)


def _tc_xw_body(x_ref, w_ref, xw_ref):
    xw_ref[...] = jnp.dot(x_ref[...], w_ref[...],
                          preferred_element_type=jnp.float32, precision=_P)


def _tc_xw(x, W):
    return pl.pallas_call(
        _tc_xw_body,
        grid=(NB,),
        in_specs=[
            pl.BlockSpec((BLK, D), lambda i: (i, 0)),
            pl.BlockSpec((D, D), lambda i: (0, 0)),
        ],
        out_specs=pl.BlockSpec((BLK, D), lambda i: (i, 0)),
        out_shape=jax.ShapeDtypeStruct((N, D), jnp.float32),
    )(x, W)


def _tc_y_body(xw_ref, degp_ref, y_ref, dinv_ref):
    deg = jnp.sum(degp_ref[0], axis=1, keepdims=True) + 1.0
    dinv = jax.lax.rsqrt(deg)
    y_ref[...] = xw_ref[...] * dinv
    dinv_ref[...] = dinv


def _tc_y(xw, degp):
    return pl.pallas_call(
        _tc_y_body,
        grid=(NB,),
        in_specs=[
            pl.BlockSpec((BLK, D), lambda i: (i, 0)),
            pl.BlockSpec((1, BLK, NW), lambda i: (i, 0, 0)),
        ],
        out_specs=[
            pl.BlockSpec((BLK, D), lambda i: (i, 0)),
            pl.BlockSpec((BLK, 1), lambda i: (i, 0)),
        ],
        out_shape=[
            jax.ShapeDtypeStruct((N, D), jnp.float32),
            jax.ShapeDtypeStruct((N, 1), jnp.float32),
        ],
    )(xw, degp)


def _tc_tail_body(aggp_ref, y_ref, dinv_ref, bgcn_ref, batch_ref,
                  wih_ref, whh_ref, bih_ref, bhh_ref, wfc_ref, bfc_ref,
                  out_ref, psum_ref, cnt_ref, hs_ref):
    i = pl.program_id(0)

    @pl.when(i == 0)
    def _():
        psum_ref[...] = jnp.zeros_like(psum_ref)
        cnt_ref[...] = jnp.zeros_like(cnt_ref)

    agg = aggp_ref[0] + aggp_ref[1]
    h = (agg + y_ref[...]) * dinv_ref[...] + bgcn_ref[...]
    h = jnp.maximum(h, 0.0)
    ids = batch_ref[0]
    onehot = (jax.lax.broadcasted_iota(jnp.int32, (G, BLK), 0) == ids
              ).astype(jnp.float32)
    psum_ref[...] += jnp.dot(onehot, h, preferred_element_type=jnp.float32,
                             precision=_P)
    cnt_ref[:, 0:1] += jnp.sum(onehot, axis=1, keepdims=True)

    @pl.when(i == NB - 1)
    def _():
        pooled = psum_ref[...] / jnp.maximum(cnt_ref[:, 0:1], 1.0)
        bias = bih_ref[...] + bhh_ref[...]
        xg = jnp.dot(pooled, wih_ref[...], preferred_element_type=jnp.float32,
                     precision=_P) + bias
        hh = jnp.zeros((1, HH), jnp.float32)
        cc = jnp.zeros((1, HH), jnp.float32)
        for t in range(G):
            g = xg[t:t + 1, :] + jnp.dot(hh, whh_ref[...],
                                         preferred_element_type=jnp.float32,
                                         precision=_P)
            ig = jax.nn.sigmoid(g[:, 0:HH])
            fg = jax.nn.sigmoid(g[:, HH:2 * HH])
            gg = jnp.tanh(g[:, 2 * HH:3 * HH])
            og = jax.nn.sigmoid(g[:, 3 * HH:4 * HH])
            cc = fg * cc + ig * gg
            hh = og * jnp.tanh(cc)
            hs_ref[t:t + 1, :] = hh
        out_ref[...] = jnp.dot(hs_ref[...], wfc_ref[...],
                               preferred_element_type=jnp.float32,
                               precision=_P) + bfc_ref[...]


def _tc_tail(aggp, y, dinv, bgcn, batch3, wihT, whhT, bih, bhh, wfcT, bfc):
    return pl.pallas_call(
        _tc_tail_body,
        grid=(NB,),
        in_specs=[
            pl.BlockSpec((NC, BLK, D), lambda i: (0, i, 0)),
            pl.BlockSpec((BLK, D), lambda i: (i, 0)),
            pl.BlockSpec((BLK, 1), lambda i: (i, 0)),
            pl.BlockSpec((1, D), lambda i: (0, 0)),
            pl.BlockSpec((1, 1, BLK), lambda i: (i, 0, 0)),
            pl.BlockSpec((HH, 4 * HH), lambda i: (0, 0)),
            pl.BlockSpec((HH, 4 * HH), lambda i: (0, 0)),
            pl.BlockSpec((1, 4 * HH), lambda i: (0, 0)),
            pl.BlockSpec((1, 4 * HH), lambda i: (0, 0)),
            pl.BlockSpec((HH, C), lambda i: (0, 0)),
            pl.BlockSpec((1, C), lambda i: (0, 0)),
        ],
        out_specs=pl.BlockSpec((G, C), lambda i: (0, 0)),
        out_shape=jax.ShapeDtypeStruct((G, C), jnp.float32),
        scratch_shapes=[
            pltpu.VMEM((G, D), jnp.float32),
            pltpu.VMEM((G, D), jnp.float32),
            pltpu.VMEM((G, HH), jnp.float32),
        ],
    )(aggp, y, dinv, bgcn, batch3, wihT, whhT, bih, bhh, wfcT, bfc)


def kernel(x, edge_index, batch, W_gcn, b_gcn, W_ih, W_hh, b_ih, b_hh,
           W_fc, b_fc):
    src = edge_index[0].astype(jnp.int32)
    dst = edge_index[1].astype(jnp.int32)

    xw = _tc_xw(x, W_gcn)
    deg_parts = _sc_degree(dst)
    degp = deg_parts[:, :N].T.reshape(NB, BLK, NW)
    y, dinv = _tc_y(xw, degp)

    agg = _sc_scatter(y, src, dst.reshape(NW, NCHUNK, B_E))
    aggp = agg.reshape(NC, N, D)

    batch3 = batch.astype(jnp.int32).reshape(NB, 1, BLK)
    out = _tc_tail(aggp, y, dinv, b_gcn.reshape(1, D), batch3,
                   W_ih.T, W_hh.T, b_ih.reshape(1, 4 * HH),
                   b_hh.reshape(1, 4 * HH), W_fc.T, b_fc.reshape(1, C))
    return out


# speedup vs baseline: 1.0505x; 1.0505x over previous
import functools

import jax
import jax.numpy as jnp
from jax import lax
from jax.experimental import pallas as pl
from jax.experimental.pallas import tpu as pltpu
from jax.experimental.pallas import tpu_sc as plsc

N = 10000
E = 320000
D = 128
HH = 128
C = 10
G = 64

NC = 2
NS = 16
NW = NC * NS
L = 16

EPT = E // NW
B_E = 80
NCHUNK = EPT // B_E
NPAD = 10240
RPT = 640
RPT_LAST = N - (NS - 1) * RPT

NBUF = 2

BLK = 1000
NB = N // BLK

_P = jax.lax.Precision.HIGHEST


def _sc_degree(dst_flat):
    mesh = plsc.VectorSubcoreMesh(core_axis_name="c", subcore_axis_name="s")

    @functools.partial(
        pl.kernel,
        out_type=jax.ShapeDtypeStruct((NW, NPAD), jnp.float32),
        mesh=mesh,
        scratch_types=[
            pltpu.VMEM((EPT,), jnp.int32),
            pltpu.VMEM((NPAD,), jnp.float32),
        ],
        compiler_params=pltpu.CompilerParams(needs_layout_passes=False),
    )
    def k(dst_hbm, out_hbm, dst_v, deg_v):
        cid = lax.axis_index("c")
        sid = lax.axis_index("s")
        w = cid * NS + sid
        pltpu.sync_copy(dst_hbm.at[pl.ds(w * EPT, EPT)], dst_v)

        zeros16 = jnp.zeros((L,), jnp.float32)

        def zbody(i, carry):
            deg_v[pl.ds(i * L, L)] = zeros16
            return carry

        lax.fori_loop(0, NPAD // L, zbody, 0)

        ones16 = jnp.ones((L,), jnp.float32)

        def sbody(i, carry):
            idx = dst_v[pl.ds(i * L, L)]
            plsc.addupdate_scatter(deg_v, [idx], ones16)
            return carry

        lax.fori_loop(0, EPT // L, sbody, 0)
        pltpu.sync_copy(deg_v, out_hbm.at[w])

    return k(dst_flat)


def _sc_scatter(y, src_flat, dst3):
    mesh = plsc.VectorSubcoreMesh(core_axis_name="c", subcore_axis_name="s")

    @functools.partial(
        pl.kernel,
        out_type=jax.ShapeDtypeStruct((NC * N, D), jnp.float32),
        mesh=mesh,
        scratch_types=[
            pltpu.VMEM((EPT,), jnp.int32),
            pltpu.VMEM((NCHUNK, B_E), jnp.int32),
            [pltpu.VMEM((B_E, D), jnp.float32) for _ in range(NBUF)],
            pltpu.VMEM_SHARED((N, D), jnp.float32),
            [pltpu.SemaphoreType.DMA for _ in range(NBUF)],
        ],
    )
    def k(y_hbm, src_hbm, dst_hbm, out_hbm, src_v, dst_v, rows_v,
          agg_sh, sems):
        cid = lax.axis_index("c")
        sid = lax.axis_index("s")
        w = cid * NS + sid

        pltpu.sync_copy(src_hbm.at[pl.ds(w * EPT, EPT)], src_v)
        pltpu.sync_copy(dst_hbm.at[w], dst_v)

        zeros16 = jnp.zeros((L,), jnp.float32)

        def zb(i, carry):
            for j in range(D // L):
                rows_v[0][i, pl.ds(j * L, L)] = zeros16
            return carry

        lax.fori_loop(0, B_E, zb, 0)

        def zs(i, carry):
            pltpu.sync_copy(rows_v[0],
                            agg_sh.at[pl.ds(sid * RPT + i * B_E, B_E)])
            return carry

        @pl.when(sid < NS - 1)
        def _():
            lax.fori_loop(0, RPT // B_E, zs, 0)

        @pl.when(sid == NS - 1)
        def _():
            lax.fori_loop(0, RPT_LAST // B_E, zs, 0)

        def gather(c, b):
            pltpu.async_copy(y_hbm.at[src_v.at[pl.ds(c * B_E, B_E)]],
                             rows_v[b], sems[b])

        for b in range(NBUF):
            gather(b, b)
        plsc.subcore_barrier()

        def wait_scatter(c, b):
            pltpu.make_async_copy(y_hbm.at[src_v.at[pl.ds(c * B_E, B_E)]],
                                  rows_v[b], sems[b]).wait()
            pltpu.sync_copy(rows_v[b], agg_sh.at[dst_v.at[c]], add=True)

        def body(i, carry):
            for b in range(NBUF):
                c = i * NBUF + b
                wait_scatter(c, b)
                gather(c + NBUF, b)
            return carry

        steady = (NCHUNK - NBUF) // NBUF
        lax.fori_loop(0, steady, body, 0)
        for b in range(NBUF):
            wait_scatter(steady * NBUF + b, b)
        for c in range((steady + 1) * NBUF, NCHUNK):
            b = c % NBUF
            gather(c, b)
            wait_scatter(c, b)
        plsc.subcore_barrier()

        @pl.when(sid < NS - 1)
        def _():
            pltpu.sync_copy(agg_sh.at[pl.ds(sid * RPT, RPT)],
                            out_hbm.at[pl.ds(cid * N + sid * RPT, RPT)])

        @pl.when(sid == NS - 1)
        def _():
            pltpu.sync_copy(
                agg_sh.at[pl.ds(sid * RPT, RPT_LAST)],
                out_hbm.at[pl.ds(cid * N + sid * RPT, RPT_LAST)])

    return k(y, src_flat, dst3)


def _tc_y_body(x_ref, w_ref, degp_ref, y_ref, dinv_ref):
    deg = jnp.sum(degp_ref[0], axis=1, keepdims=True) + 1.0
    dinv = jax.lax.rsqrt(deg)
    xw = jnp.dot(x_ref[...], w_ref[...], preferred_element_type=jnp.float32,
                 precision=_P)
    y_ref[...] = xw * dinv
    dinv_ref[...] = dinv


def _tc_y(x, W, degp):
    return pl.pallas_call(
        _tc_y_body,
        grid=(NB,),
        in_specs=[
            pl.BlockSpec((BLK, D), lambda i: (i, 0)),
            pl.BlockSpec((D, D), lambda i: (0, 0)),
            pl.BlockSpec((1, BLK, NW), lambda i: (i, 0, 0)),
        ],
        out_specs=[
            pl.BlockSpec((BLK, D), lambda i: (i, 0)),
            pl.BlockSpec((BLK, 1), lambda i: (i, 0)),
        ],
        out_shape=[
            jax.ShapeDtypeStruct((N, D), jnp.float32),
            jax.ShapeDtypeStruct((N, 1), jnp.float32),
        ],
    )(x, W, degp)


def _tc_tail_body(aggp_ref, y_ref, dinv_ref, bgcn_ref, batch_ref,
                  wih_ref, whh_ref, bih_ref, bhh_ref, wfc_ref, bfc_ref,
                  out_ref, psum_ref, cnt_ref, hs_ref):
    i = pl.program_id(0)

    @pl.when(i == 0)
    def _():
        psum_ref[...] = jnp.zeros_like(psum_ref)
        cnt_ref[...] = jnp.zeros_like(cnt_ref)

    agg = aggp_ref[0] + aggp_ref[1]
    h = (agg + y_ref[...]) * dinv_ref[...] + bgcn_ref[...]
    h = jnp.maximum(h, 0.0)
    ids = batch_ref[0]
    onehot = (jax.lax.broadcasted_iota(jnp.int32, (G, BLK), 0) == ids
              ).astype(jnp.float32)
    psum_ref[...] += jnp.dot(onehot, h, preferred_element_type=jnp.float32,
                             precision=_P)
    cnt_ref[:, 0:1] += jnp.sum(onehot, axis=1, keepdims=True)

    @pl.when(i == NB - 1)
    def _():
        pooled = psum_ref[...] / jnp.maximum(cnt_ref[:, 0:1], 1.0)
        bias = bih_ref[...] + bhh_ref[...]
        xg = jnp.dot(pooled, wih_ref[...], preferred_element_type=jnp.float32,
                     precision=_P) + bias
        hh = jnp.zeros((1, HH), jnp.float32)
        cc = jnp.zeros((1, HH), jnp.float32)
        for t in range(G):
            g = xg[t:t + 1, :] + jnp.dot(hh, whh_ref[...],
                                         preferred_element_type=jnp.float32)
            ig = jax.nn.sigmoid(g[:, 0:HH])
            fg = jax.nn.sigmoid(g[:, HH:2 * HH])
            gg = jnp.tanh(g[:, 2 * HH:3 * HH])
            og = jax.nn.sigmoid(g[:, 3 * HH:4 * HH])
            cc = fg * cc + ig * gg
            hh = og * jnp.tanh(cc)
            hs_ref[t:t + 1, :] = hh
        out_ref[...] = jnp.dot(hs_ref[...], wfc_ref[...],
                               preferred_element_type=jnp.float32,
                               precision=_P) + bfc_ref[...]


def _tc_tail(aggp, y, dinv, bgcn, batch3, wihT, whhT, bih, bhh, wfcT, bfc):
    return pl.pallas_call(
        _tc_tail_body,
        grid=(NB,),
        in_specs=[
            pl.BlockSpec((NC, BLK, D), lambda i: (0, i, 0)),
            pl.BlockSpec((BLK, D), lambda i: (i, 0)),
            pl.BlockSpec((BLK, 1), lambda i: (i, 0)),
            pl.BlockSpec((1, D), lambda i: (0, 0)),
            pl.BlockSpec((1, 1, BLK), lambda i: (i, 0, 0)),
            pl.BlockSpec((HH, 4 * HH), lambda i: (0, 0)),
            pl.BlockSpec((HH, 4 * HH), lambda i: (0, 0)),
            pl.BlockSpec((1, 4 * HH), lambda i: (0, 0)),
            pl.BlockSpec((1, 4 * HH), lambda i: (0, 0)),
            pl.BlockSpec((HH, C), lambda i: (0, 0)),
            pl.BlockSpec((1, C), lambda i: (0, 0)),
        ],
        out_specs=pl.BlockSpec((G, C), lambda i: (0, 0)),
        out_shape=jax.ShapeDtypeStruct((G, C), jnp.float32),
        scratch_shapes=[
            pltpu.VMEM((G, D), jnp.float32),
            pltpu.VMEM((G, D), jnp.float32),
            pltpu.VMEM((G, HH), jnp.float32),
        ],
    )(aggp, y, dinv, bgcn, batch3, wihT, whhT, bih, bhh, wfcT, bfc)


def kernel(x, edge_index, batch, W_gcn, b_gcn, W_ih, W_hh, b_ih, b_hh,
           W_fc, b_fc):
    src = edge_index[0].astype(jnp.int32)
    dst = edge_index[1].astype(jnp.int32)

    deg_parts = _sc_degree(dst)
    degp = deg_parts[:, :N].T.reshape(NB, BLK, NW)
    y, dinv = _tc_y(x, W_gcn, degp)

    agg = _sc_scatter(y, src, dst.reshape(NW, NCHUNK, B_E))
    aggp = agg.reshape(NC, N, D)

    batch3 = batch.astype(jnp.int32).reshape(NB, 1, BLK)
    out = _tc_tail(aggp, y, dinv, b_gcn.reshape(1, D), batch3,
                   W_ih.T, W_hh.T, b_ih.reshape(1, 4 * HH),
                   b_hh.reshape(1, 4 * HH), W_fc.T, b_fc.reshape(1, C))
    return out


# speedup vs baseline: 1.0758x; 1.0240x over previous
import functools

import jax
import jax.numpy as jnp
from jax import lax
from jax.experimental import pallas as pl
from jax.experimental.pallas import tpu as pltpu
from jax.experimental.pallas import tpu_sc as plsc

N = 10000
E = 320000
D = 128
HH = 128
C = 10
G = 64

NC = 2
NS = 16
NW = NC * NS
L = 16

EPT = E // NW
B_E = 80
NCHUNK = EPT // B_E
NPAD = 10240
RPT = 640
RPT_LAST = N - (NS - 1) * RPT

NBUF = 2

BLK = 1000
NB = N // BLK


def _sc_degree(dst_flat):
    mesh = plsc.VectorSubcoreMesh(core_axis_name="c", subcore_axis_name="s")

    @functools.partial(
        pl.kernel,
        out_type=jax.ShapeDtypeStruct((NW, NPAD), jnp.float32),
        mesh=mesh,
        scratch_types=[
            pltpu.VMEM((EPT,), jnp.int32),
            pltpu.VMEM((NPAD,), jnp.float32),
        ],
        compiler_params=pltpu.CompilerParams(needs_layout_passes=False),
    )
    def k(dst_hbm, out_hbm, dst_v, deg_v):
        cid = lax.axis_index("c")
        sid = lax.axis_index("s")
        w = cid * NS + sid
        pltpu.sync_copy(dst_hbm.at[pl.ds(w * EPT, EPT)], dst_v)

        zeros16 = jnp.zeros((L,), jnp.float32)

        def zbody(i, carry):
            deg_v[pl.ds(i * L, L)] = zeros16
            return carry

        lax.fori_loop(0, NPAD // L, zbody, 0)

        ones16 = jnp.ones((L,), jnp.float32)

        def sbody(i, carry):
            idx = dst_v[pl.ds(i * L, L)]
            plsc.addupdate_scatter(deg_v, [idx], ones16)
            return carry

        lax.fori_loop(0, EPT // L, sbody, 0)
        pltpu.sync_copy(deg_v, out_hbm.at[w])

    return k(dst_flat)


def _sc_scatter(y, src_flat, dst3):
    mesh = plsc.VectorSubcoreMesh(core_axis_name="c", subcore_axis_name="s")

    @functools.partial(
        pl.kernel,
        out_type=jax.ShapeDtypeStruct((NC * N, D), jnp.float32),
        mesh=mesh,
        scratch_types=[
            pltpu.VMEM((EPT,), jnp.int32),
            pltpu.VMEM((NCHUNK, B_E), jnp.int32),
            [pltpu.VMEM((B_E, D), jnp.float32) for _ in range(NBUF)],
            pltpu.VMEM_SHARED((N, D), jnp.float32),
            [pltpu.SemaphoreType.DMA for _ in range(NBUF)],
        ],
    )
    def k(y_hbm, src_hbm, dst_hbm, out_hbm, src_v, dst_v, rows_v,
          agg_sh, sems):
        cid = lax.axis_index("c")
        sid = lax.axis_index("s")
        w = cid * NS + sid

        pltpu.sync_copy(src_hbm.at[pl.ds(w * EPT, EPT)], src_v)
        pltpu.sync_copy(dst_hbm.at[w], dst_v)

        zeros16 = jnp.zeros((L,), jnp.float32)

        def zb(i, carry):
            for j in range(D // L):
                rows_v[0][i, pl.ds(j * L, L)] = zeros16
            return carry

        lax.fori_loop(0, B_E, zb, 0)

        def zs(i, carry):
            pltpu.sync_copy(rows_v[0],
                            agg_sh.at[pl.ds(sid * RPT + i * B_E, B_E)])
            return carry

        @pl.when(sid < NS - 1)
        def _():
            lax.fori_loop(0, RPT // B_E, zs, 0)

        @pl.when(sid == NS - 1)
        def _():
            lax.fori_loop(0, RPT_LAST // B_E, zs, 0)

        def gather(c, b):
            pltpu.async_copy(y_hbm.at[src_v.at[pl.ds(c * B_E, B_E)]],
                             rows_v[b], sems[b])

        for b in range(NBUF):
            gather(b, b)
        plsc.subcore_barrier()

        def wait_scatter(c, b):
            pltpu.make_async_copy(y_hbm.at[src_v.at[pl.ds(c * B_E, B_E)]],
                                  rows_v[b], sems[b]).wait()
            pltpu.sync_copy(rows_v[b], agg_sh.at[dst_v.at[c]], add=True)

        def body(i, carry):
            for b in range(NBUF):
                c = i * NBUF + b
                wait_scatter(c, b)
                gather(c + NBUF, b)
            return carry

        steady = (NCHUNK - NBUF) // NBUF
        lax.fori_loop(0, steady, body, 0)
        for b in range(NBUF):
            wait_scatter(steady * NBUF + b, b)
        for c in range((steady + 1) * NBUF, NCHUNK):
            b = c % NBUF
            gather(c, b)
            wait_scatter(c, b)
        plsc.subcore_barrier()

        @pl.when(sid < NS - 1)
        def _():
            pltpu.sync_copy(agg_sh.at[pl.ds(sid * RPT, RPT)],
                            out_hbm.at[pl.ds(cid * N + sid * RPT, RPT)])

        @pl.when(sid == NS - 1)
        def _():
            pltpu.sync_copy(
                agg_sh.at[pl.ds(sid * RPT, RPT_LAST)],
                out_hbm.at[pl.ds(cid * N + sid * RPT, RPT_LAST)])

    return k(y, src_flat, dst3)


def _tc_y_body(x_ref, w_ref, degp_ref, y_ref, dinv_ref):
    deg = jnp.sum(degp_ref[0], axis=1, keepdims=True) + 1.0
    dinv = jax.lax.rsqrt(deg)
    xw = jnp.dot(x_ref[...], w_ref[...], preferred_element_type=jnp.float32)
    y_ref[...] = xw * dinv
    dinv_ref[...] = dinv


def _tc_y(x, W, degp):
    return pl.pallas_call(
        _tc_y_body,
        grid=(NB,),
        in_specs=[
            pl.BlockSpec((BLK, D), lambda i: (i, 0)),
            pl.BlockSpec((D, D), lambda i: (0, 0)),
            pl.BlockSpec((1, BLK, NW), lambda i: (i, 0, 0)),
        ],
        out_specs=[
            pl.BlockSpec((BLK, D), lambda i: (i, 0)),
            pl.BlockSpec((BLK, 1), lambda i: (i, 0)),
        ],
        out_shape=[
            jax.ShapeDtypeStruct((N, D), jnp.float32),
            jax.ShapeDtypeStruct((N, 1), jnp.float32),
        ],
    )(x, W, degp)


def _tc_tail_body(aggp_ref, y_ref, dinv_ref, bgcn_ref, batch_ref,
                  wih_ref, whh_ref, bih_ref, bhh_ref, wfc_ref, bfc_ref,
                  out_ref, psum_ref, cnt_ref, hs_ref):
    i = pl.program_id(0)

    @pl.when(i == 0)
    def _():
        psum_ref[...] = jnp.zeros_like(psum_ref)
        cnt_ref[...] = jnp.zeros_like(cnt_ref)

    agg = aggp_ref[0] + aggp_ref[1]
    h = (agg + y_ref[...]) * dinv_ref[...] + bgcn_ref[...]
    h = jnp.maximum(h, 0.0)
    ids = batch_ref[0]
    onehot = (jax.lax.broadcasted_iota(jnp.int32, (G, BLK), 0) == ids
              ).astype(jnp.float32)
    psum_ref[...] += jnp.dot(onehot, h, preferred_element_type=jnp.float32)
    cnt_ref[:, 0:1] += jnp.sum(onehot, axis=1, keepdims=True)

    @pl.when(i == NB - 1)
    def _():
        pooled = psum_ref[...] / jnp.maximum(cnt_ref[:, 0:1], 1.0)
        bias = bih_ref[...] + bhh_ref[...]
        xg = jnp.dot(pooled, wih_ref[...],
                     preferred_element_type=jnp.float32) + bias
        hh = jnp.zeros((1, HH), jnp.float32)
        cc = jnp.zeros((1, HH), jnp.float32)
        for t in range(G):
            g = xg[t:t + 1, :] + jnp.dot(hh, whh_ref[...],
                                         preferred_element_type=jnp.float32)
            ig = jax.nn.sigmoid(g[:, 0:HH])
            fg = jax.nn.sigmoid(g[:, HH:2 * HH])
            gg = jnp.tanh(g[:, 2 * HH:3 * HH])
            og = jax.nn.sigmoid(g[:, 3 * HH:4 * HH])
            cc = fg * cc + ig * gg
            hh = og * jnp.tanh(cc)
            hs_ref[t:t + 1, :] = hh
        out_ref[...] = jnp.dot(hs_ref[...], wfc_ref[...],
                               preferred_element_type=jnp.float32) + bfc_ref[...]


def _tc_tail(aggp, y, dinv, bgcn, batch3, wihT, whhT, bih, bhh, wfcT, bfc):
    return pl.pallas_call(
        _tc_tail_body,
        grid=(NB,),
        in_specs=[
            pl.BlockSpec((NC, BLK, D), lambda i: (0, i, 0)),
            pl.BlockSpec((BLK, D), lambda i: (i, 0)),
            pl.BlockSpec((BLK, 1), lambda i: (i, 0)),
            pl.BlockSpec((1, D), lambda i: (0, 0)),
            pl.BlockSpec((1, 1, BLK), lambda i: (i, 0, 0)),
            pl.BlockSpec((HH, 4 * HH), lambda i: (0, 0)),
            pl.BlockSpec((HH, 4 * HH), lambda i: (0, 0)),
            pl.BlockSpec((1, 4 * HH), lambda i: (0, 0)),
            pl.BlockSpec((1, 4 * HH), lambda i: (0, 0)),
            pl.BlockSpec((HH, C), lambda i: (0, 0)),
            pl.BlockSpec((1, C), lambda i: (0, 0)),
        ],
        out_specs=pl.BlockSpec((G, C), lambda i: (0, 0)),
        out_shape=jax.ShapeDtypeStruct((G, C), jnp.float32),
        scratch_shapes=[
            pltpu.VMEM((G, D), jnp.float32),
            pltpu.VMEM((G, D), jnp.float32),
            pltpu.VMEM((G, HH), jnp.float32),
        ],
    )(aggp, y, dinv, bgcn, batch3, wihT, whhT, bih, bhh, wfcT, bfc)


def kernel(x, edge_index, batch, W_gcn, b_gcn, W_ih, W_hh, b_ih, b_hh,
           W_fc, b_fc):
    src = edge_index[0].astype(jnp.int32)
    dst = edge_index[1].astype(jnp.int32)

    deg_parts = _sc_degree(dst)
    degp = deg_parts[:, :N].T.reshape(NB, BLK, NW)
    y, dinv = _tc_y(x, W_gcn, degp)

    agg = _sc_scatter(y, src, dst.reshape(NW, NCHUNK, B_E))
    aggp = agg.reshape(NC, N, D)

    batch3 = batch.astype(jnp.int32).reshape(NB, 1, BLK)
    out = _tc_tail(aggp, y, dinv, b_gcn.reshape(1, D), batch3,
                   W_ih.T, W_hh.T, b_ih.reshape(1, 4 * HH),
                   b_hh.reshape(1, 4 * HH), W_fc.T, b_fc.reshape(1, C))
    return out
